# baseline (speedup 1.0000x reference)
import jax
import jax.numpy as jnp
from jax.experimental import pallas as pl

_BPS = 16


def _chamfer_body(p_ref, out_ref):
    for j in range(_BPS):
        at = p_ref[j, :4, :]
        bt = p_ref[j, 4:, :]
        row_a = jax.lax.broadcasted_iota(jnp.int32, at.shape, 0)
        a_aug = jnp.where(row_a == 3, 1.0, -2.0 * at)
        b2 = jnp.sum(bt * bt, axis=0, keepdims=True)
        row_b = jax.lax.broadcasted_iota(jnp.int32, bt.shape, 0)
        bt_aug = jnp.where(row_b == 3, b2, bt)
        d = jax.lax.dot_general(
            a_aug, bt_aug, (((0,), (0,)), ((), ())),
            preferred_element_type=jnp.float32)
        m = jnp.min(d, axis=1)
        a2 = jnp.sum(at * at, axis=0)
        loss = jnp.mean(a2 + m)
        total = loss if j == 0 else total + loss
    out_ref[...] = jnp.broadcast_to(total, out_ref.shape)


def kernel(adv_pc, ori_pc):
    B, K, _ = adv_pc.shape
    pts = jnp.concatenate(
        [adv_pc, jnp.zeros((B, K, 1), jnp.float32),
         ori_pc, jnp.zeros((B, K, 1), jnp.float32)], axis=2)
    p = pts.transpose(0, 2, 1)
    steps = B // _BPS
    out = pl.pallas_call(
        _chamfer_body,
        grid=(steps,),
        in_specs=[pl.BlockSpec((_BPS, 8, K), lambda b: (b, 0, 0))],
        out_specs=pl.BlockSpec((1, 1, 128), lambda b: (b, 0, 0)),
        out_shape=jax.ShapeDtypeStruct((steps, 1, 128), jnp.float32),
    )(p)
    return jnp.sum(out[:, 0, 0]) / B


# speedup vs baseline: 1.0994x; 1.0994x over previous
import jax
import jax.numpy as jnp
from jax.experimental import pallas as pl

_BPS = 8


def _chamfer_body(p_ref, out_ref):
    for j in range(_BPS):
        at = p_ref[j, :4, :]
        bt = p_ref[j, 4:, :]
        row_a = jax.lax.broadcasted_iota(jnp.int32, at.shape, 0)
        a_aug = jnp.where(row_a == 3, 1.0, -2.0 * at)
        b2 = jnp.sum(bt * bt, axis=0, keepdims=True)
        row_b = jax.lax.broadcasted_iota(jnp.int32, bt.shape, 0)
        bt_aug = jnp.where(row_b == 3, b2, bt)
        d = jax.lax.dot_general(
            a_aug, bt_aug, (((0,), (0,)), ((), ())),
            preferred_element_type=jnp.float32)
        m = jnp.min(d, axis=1)
        a2 = jnp.sum(at * at, axis=0)
        loss = jnp.mean(a2 + m)
        total = loss if j == 0 else total + loss
    out_ref[...] = jnp.broadcast_to(total, out_ref.shape)


def kernel(adv_pc, ori_pc):
    B, K, _ = adv_pc.shape
    pts = jnp.concatenate(
        [adv_pc, jnp.zeros((B, K, 1), jnp.float32),
         ori_pc, jnp.zeros((B, K, 1), jnp.float32)], axis=2)
    p = pts.transpose(0, 2, 1)
    steps = B // _BPS
    out = pl.pallas_call(
        _chamfer_body,
        grid=(steps,),
        in_specs=[pl.BlockSpec((_BPS, 8, K), lambda b: (b, 0, 0))],
        out_specs=pl.BlockSpec((1, 1, 128), lambda b: (b, 0, 0)),
        out_shape=jax.ShapeDtypeStruct((steps, 1, 128), jnp.float32),
    )(p)
    return jnp.sum(out[:, 0, 0]) / B
